# Initial kernel scaffold; baseline (speedup 1.0000x reference)
import jax
import jax.numpy as jnp
from jax.experimental import pallas as pl


def kernel(source_feats, target_feats):
    raise NotImplementedError("write your pallas kernel here")


# speedup vs baseline: 1.0350x; 1.0350x over previous
import functools

import jax
import jax.numpy as jnp
from jax.experimental import pallas as pl
from jax.experimental.pallas import tpu as pltpu
from jax.experimental.pallas import tpu_sc as plsc

K_NN = 4
BT = 512


def topk_body(src_ref, tgt_ref, idx_out_ref, srcn_ref, rv_ref, ri_ref, *, t_total):
    bt = pl.program_id(0)
    nblk = pl.num_programs(0)
    q, _ = src_ref.shape
    btn = tgt_ref.shape[0]

    @pl.when(bt == 0)
    def _init():
        s = src_ref[...]
        n = jnp.sqrt(jnp.sum(s * s, axis=1, keepdims=True)) + 1e-8
        srcn_ref[...] = (s / n).astype(jnp.bfloat16)
        rv_ref[...] = jnp.full((q, K_NN), -jnp.inf, jnp.float32)
        ri_ref[...] = jnp.zeros((q, K_NN), jnp.int32)

    tb = tgt_ref[...]
    tn = jnp.sqrt(jnp.sum(tb * tb, axis=1, keepdims=True)) + 1e-8
    tbn = (tb / tn).astype(jnp.bfloat16)
    sim = jax.lax.dot_general(
        srcn_ref[...], tbn,
        dimension_numbers=(((1,), (1,)), ((), ())),
        preferred_element_type=jnp.float32,
    )

    col = jax.lax.broadcasted_iota(jnp.int32, (q, btn), 1)
    sim = jnp.where(bt * btn + col < t_total, sim, -jnp.inf)

    big = jnp.int32(2**30)
    vals, idxs = [], []
    s = sim
    for _ in range(K_NN):
        m = jnp.max(s, axis=1, keepdims=True)
        am = jnp.min(jnp.where(s == m, col, big), axis=1, keepdims=True)
        vals.append(m)
        idxs.append(am + bt * btn)
        s = jnp.where(col == am, -jnp.inf, s)

    cv = jnp.concatenate([rv_ref[...]] + vals, axis=1)
    ci = jnp.concatenate([ri_ref[...]] + idxs, axis=1)
    slot = jax.lax.broadcasted_iota(jnp.int32, (q, 2 * K_NN), 1)
    nv, ni = [], []
    for _ in range(K_NN):
        m = jnp.max(cv, axis=1, keepdims=True)
        am = jnp.min(jnp.where(cv == m, slot, big), axis=1, keepdims=True)
        nv.append(m)
        ni.append(jnp.sum(jnp.where(slot == am, ci, 0), axis=1, keepdims=True))
        cv = jnp.where(slot == am, -jnp.inf, cv)
    rv_ref[...] = jnp.concatenate(nv, axis=1)
    ri_ref[...] = jnp.concatenate(ni, axis=1)

    @pl.when(bt == nblk - 1)
    def _emit():
        chunks = idx_out_ref.shape[1] // K_NN
        ri = ri_ref[...]
        rep = jnp.concatenate([ri] * chunks, axis=1)
        c = jax.lax.broadcasted_iota(jnp.int32, rep.shape, 1) // K_NN
        idx_out_ref[...] = rep * chunks + c


def topk_indices(source_feats, target_feats, interpret=False):
    q, d = source_feats.shape
    t = target_feats.shape[0]
    chunks = d // 128
    nblk = pl.cdiv(t, BT)
    return pl.pallas_call(
        functools.partial(topk_body, t_total=t),
        grid=(nblk,),
        in_specs=[
            pl.BlockSpec((q, d), lambda i: (0, 0)),
            pl.BlockSpec((BT, d), lambda i: (i, 0)),
        ],
        out_specs=pl.BlockSpec((q, K_NN * chunks), lambda i: (0, 0)),
        out_shape=jax.ShapeDtypeStruct((q, K_NN * chunks), jnp.int32),
        scratch_shapes=[
            pltpu.VMEM((q, d), jnp.bfloat16),
            pltpu.VMEM((q, K_NN), jnp.float32),
            pltpu.VMEM((q, K_NN), jnp.int32),
        ],
        compiler_params=pltpu.CompilerParams(
            dimension_semantics=("arbitrary",),
        ),
        interpret=interpret,
    )(source_feats, target_feats)


def gather_mean(target_feats, idx):
    q = idx.shape[0]
    t, d = target_feats.shape
    chunks = d // 128
    w = 128
    rows_out = w // K_NN
    tgt_view = target_feats.reshape(t * chunks, 128)
    idx_flat = idx.reshape(1, q * K_NN * chunks)
    mesh = plsc.VectorSubcoreMesh(core_axis_name="core", subcore_axis_name="subcore")

    @pl.kernel(
        out_type=jax.ShapeDtypeStruct((q * chunks, 128), jnp.float32),
        mesh=mesh,
        scratch_types=[pltpu.VMEM((w, 128), jnp.float32)],
    )
    def sc_kernel(tgt_hbm, idx_hbm, out_hbm, g_vmem):
        def body(i_vmem, o_vmem):
            pltpu.sync_copy(tgt_hbm.at[i_vmem.at[0]], g_vmem)

            @pl.loop(0, rows_out)
            def _row(r):
                @pl.loop(0, 128, step=16)
                def _col(c):
                    acc = (
                        g_vmem[K_NN * r, pl.ds(c, 16)]
                        + g_vmem[K_NN * r + 1, pl.ds(c, 16)]
                        + g_vmem[K_NN * r + 2, pl.ds(c, 16)]
                        + g_vmem[K_NN * r + 3, pl.ds(c, 16)]
                    )
                    o_vmem[r, pl.ds(c, 16)] = acc * 0.25

        pltpu.emit_pipeline(
            body,
            grid=(q * K_NN * chunks // w,),
            in_specs=[pl.BlockSpec((1, w), lambda i: (0, i))],
            out_specs=[pl.BlockSpec((rows_out, 128), lambda i: (i, 0))],
            core_axis_name=("core", "subcore"),
            dimension_semantics=(pltpu.PARALLEL,),
        )(idx_hbm, out_hbm)

    return sc_kernel(tgt_view, idx_flat).reshape(q, d)


def kernel(source_feats, target_feats):
    idx = topk_indices(source_feats, target_feats)
    return gather_mean(target_feats, idx)


# speedup vs baseline: 1.7895x; 1.7290x over previous
import functools

import jax
import jax.numpy as jnp
from jax.experimental import pallas as pl
from jax.experimental.pallas import tpu as pltpu
from jax.experimental.pallas import tpu_sc as plsc

K_NN = 4
BT = 512


LANES = 128


def topk_body(src_ref, tgt_ref, idx_out_ref, srcn_ref, pv_ref, pi_ref, *, t_total):
    bt = pl.program_id(0)
    nblk = pl.num_programs(0)
    q, _ = src_ref.shape
    btn = tgt_ref.shape[0]
    groups = btn // LANES

    @pl.when(bt == 0)
    def _init():
        s = src_ref[...]
        n = jnp.sqrt(jnp.sum(s * s, axis=1, keepdims=True)) + 1e-8
        srcn_ref[...] = (s / n).astype(jnp.bfloat16)
        pv_ref[...] = jnp.full(pv_ref.shape, -jnp.inf, jnp.float32)
        pi_ref[...] = jnp.zeros(pi_ref.shape, jnp.int32)

    tb = tgt_ref[...]
    tn = jnp.sqrt(jnp.sum(tb * tb, axis=1, keepdims=True)) + 1e-8
    tbn = (tb / tn).astype(jnp.bfloat16)
    sim = jax.lax.dot_general(
        srcn_ref[...], tbn,
        dimension_numbers=(((1,), (1,)), ((), ())),
        preferred_element_type=jnp.float32,
    )

    a = [pv_ref[:, s * LANES:(s + 1) * LANES] for s in range(K_NN)]
    ix = [pi_ref[:, s * LANES:(s + 1) * LANES] for s in range(K_NN)]
    lane = jax.lax.broadcasted_iota(jnp.int32, (q, LANES), 1)
    for g in range(groups):
        base = bt * btn + g * LANES
        x = sim[:, g * LANES:(g + 1) * LANES]
        x = jnp.where(lane < t_total - base, x, -jnp.inf)
        xi = lane + base
        c0 = x > a[0]
        c1 = x > a[1]
        c2 = x > a[2]
        c3 = x > a[3]
        a, ix = (
            [
                jnp.where(c0, x, a[0]),
                jnp.where(c0, a[0], jnp.where(c1, x, a[1])),
                jnp.where(c1, a[1], jnp.where(c2, x, a[2])),
                jnp.where(c2, a[2], jnp.where(c3, x, a[3])),
            ],
            [
                jnp.where(c0, xi, ix[0]),
                jnp.where(c0, ix[0], jnp.where(c1, xi, ix[1])),
                jnp.where(c1, ix[1], jnp.where(c2, xi, ix[2])),
                jnp.where(c2, ix[2], jnp.where(c3, xi, ix[3])),
            ],
        )
    pv_ref[...] = jnp.concatenate(a, axis=1)
    pi_ref[...] = jnp.concatenate(ix, axis=1)

    @pl.when(bt == nblk - 1)
    def _emit():
        big = jnp.int32(2**30)
        vals = jnp.concatenate(a, axis=1)
        idxs = jnp.concatenate(ix, axis=1)
        ri = []
        for _ in range(K_NN):
            m = jnp.max(vals, axis=1, keepdims=True)
            ti = jnp.min(jnp.where(vals == m, idxs, big), axis=1, keepdims=True)
            ri.append(ti)
            vals = jnp.where((vals == m) & (idxs == ti), -jnp.inf, vals)
        ri = jnp.concatenate(ri, axis=1)
        chunks = idx_out_ref.shape[1] // K_NN
        rep = jnp.concatenate([ri] * chunks, axis=1)
        c = jax.lax.broadcasted_iota(jnp.int32, rep.shape, 1) // K_NN
        idx_out_ref[...] = rep * chunks + c


def topk_indices(source_feats, target_feats, interpret=False):
    q, d = source_feats.shape
    t = target_feats.shape[0]
    chunks = d // 128
    nblk = pl.cdiv(t, BT)
    return pl.pallas_call(
        functools.partial(topk_body, t_total=t),
        grid=(nblk,),
        in_specs=[
            pl.BlockSpec((q, d), lambda i: (0, 0)),
            pl.BlockSpec((BT, d), lambda i: (i, 0)),
        ],
        out_specs=pl.BlockSpec((q, K_NN * chunks), lambda i: (0, 0)),
        out_shape=jax.ShapeDtypeStruct((q, K_NN * chunks), jnp.int32),
        scratch_shapes=[
            pltpu.VMEM((q, d), jnp.bfloat16),
            pltpu.VMEM((q, K_NN * LANES), jnp.float32),
            pltpu.VMEM((q, K_NN * LANES), jnp.int32),
        ],
        compiler_params=pltpu.CompilerParams(
            dimension_semantics=("arbitrary",),
        ),
        interpret=interpret,
    )(source_feats, target_feats)


def gather_mean(target_feats, idx):
    q = idx.shape[0]
    t, d = target_feats.shape
    chunks = d // 128
    w = 128
    rows_out = w // K_NN
    tgt_view = target_feats.reshape(t * chunks, 128)
    idx_flat = idx.reshape(1, q * K_NN * chunks)
    mesh = plsc.VectorSubcoreMesh(core_axis_name="core", subcore_axis_name="subcore")

    @pl.kernel(
        out_type=jax.ShapeDtypeStruct((q * chunks, 128), jnp.float32),
        mesh=mesh,
        scratch_types=[pltpu.VMEM((w, 128), jnp.float32)],
    )
    def sc_kernel(tgt_hbm, idx_hbm, out_hbm, g_vmem):
        def body(i_vmem, o_vmem):
            pltpu.sync_copy(tgt_hbm.at[i_vmem.at[0]], g_vmem)

            @pl.loop(0, rows_out)
            def _row(r):
                @pl.loop(0, 128, step=16)
                def _col(c):
                    acc = (
                        g_vmem[K_NN * r, pl.ds(c, 16)]
                        + g_vmem[K_NN * r + 1, pl.ds(c, 16)]
                        + g_vmem[K_NN * r + 2, pl.ds(c, 16)]
                        + g_vmem[K_NN * r + 3, pl.ds(c, 16)]
                    )
                    o_vmem[r, pl.ds(c, 16)] = acc * 0.25

        pltpu.emit_pipeline(
            body,
            grid=(q * K_NN * chunks // w,),
            in_specs=[pl.BlockSpec((1, w), lambda i: (0, i))],
            out_specs=[pl.BlockSpec((rows_out, 128), lambda i: (i, 0))],
            core_axis_name=("core", "subcore"),
            dimension_semantics=(pltpu.PARALLEL,),
        )(idx_hbm, out_hbm)

    return sc_kernel(tgt_view, idx_flat).reshape(q, d)


def kernel(source_feats, target_feats):
    idx = topk_indices(source_feats, target_feats)
    return gather_mean(target_feats, idx)


# speedup vs baseline: 1.8464x; 1.0318x over previous
import functools

import jax
import jax.numpy as jnp
from jax.experimental import pallas as pl
from jax.experimental.pallas import tpu as pltpu
from jax.experimental.pallas import tpu_sc as plsc

K_NN = 4
BT = 1024


LANES = 128


def topk_body(src_ref, tgt_ref, idx_out_ref, srcn_ref, pv_ref, pi_ref, *, t_total):
    bt = pl.program_id(0)
    nblk = pl.num_programs(0)
    q, _ = src_ref.shape
    btn = tgt_ref.shape[0]
    groups = btn // LANES

    @pl.when(bt == 0)
    def _init():
        s = src_ref[...]
        n = jnp.sqrt(jnp.sum(s * s, axis=1, keepdims=True)) + 1e-8
        srcn_ref[...] = (s / n).astype(jnp.bfloat16)
        pv_ref[...] = jnp.full(pv_ref.shape, -jnp.inf, jnp.float32)
        pi_ref[...] = jnp.zeros(pi_ref.shape, jnp.int32)

    tb = tgt_ref[...]
    tn = jnp.sqrt(jnp.sum(tb * tb, axis=1, keepdims=True)) + 1e-8
    tbn = (tb / tn).astype(jnp.bfloat16)
    sim = jax.lax.dot_general(
        srcn_ref[...], tbn,
        dimension_numbers=(((1,), (1,)), ((), ())),
        preferred_element_type=jnp.float32,
    )

    a = [pv_ref[:, s * LANES:(s + 1) * LANES] for s in range(K_NN)]
    ix = [pi_ref[:, s * LANES:(s + 1) * LANES] for s in range(K_NN)]
    lane = jax.lax.broadcasted_iota(jnp.int32, (q, LANES), 1)
    for g in range(groups):
        base = bt * btn + g * LANES
        x = sim[:, g * LANES:(g + 1) * LANES]
        x = jnp.where(lane < t_total - base, x, -jnp.inf)
        xi = lane + base
        c0 = x > a[0]
        c1 = x > a[1]
        c2 = x > a[2]
        c3 = x > a[3]
        a, ix = (
            [
                jnp.where(c0, x, a[0]),
                jnp.where(c0, a[0], jnp.where(c1, x, a[1])),
                jnp.where(c1, a[1], jnp.where(c2, x, a[2])),
                jnp.where(c2, a[2], jnp.where(c3, x, a[3])),
            ],
            [
                jnp.where(c0, xi, ix[0]),
                jnp.where(c0, ix[0], jnp.where(c1, xi, ix[1])),
                jnp.where(c1, ix[1], jnp.where(c2, xi, ix[2])),
                jnp.where(c2, ix[2], jnp.where(c3, xi, ix[3])),
            ],
        )
    for s in range(K_NN):
        pv_ref[:, s * LANES:(s + 1) * LANES] = a[s]
        pi_ref[:, s * LANES:(s + 1) * LANES] = ix[s]

    @pl.when(bt == nblk - 1)
    def _emit():
        big = jnp.int32(2**30)
        vals = jnp.concatenate(a, axis=1)
        idxs = jnp.concatenate(ix, axis=1)
        ri = []
        for _ in range(K_NN):
            m = jnp.max(vals, axis=1, keepdims=True)
            ti = jnp.min(jnp.where(vals == m, idxs, big), axis=1, keepdims=True)
            ri.append(ti)
            vals = jnp.where((vals == m) & (idxs == ti), -jnp.inf, vals)
        ri = jnp.concatenate(ri, axis=1)
        chunks = idx_out_ref.shape[1] // K_NN
        rep = jnp.concatenate([ri] * chunks, axis=1)
        c = jax.lax.broadcasted_iota(jnp.int32, rep.shape, 1) // K_NN
        idx_out_ref[...] = rep * chunks + c


def topk_indices(source_feats, target_feats, interpret=False):
    q, d = source_feats.shape
    t = target_feats.shape[0]
    chunks = d // 128
    nblk = pl.cdiv(t, BT)
    return pl.pallas_call(
        functools.partial(topk_body, t_total=t),
        grid=(nblk,),
        in_specs=[
            pl.BlockSpec((q, d), lambda i: (0, 0)),
            pl.BlockSpec((BT, d), lambda i: (i, 0)),
        ],
        out_specs=pl.BlockSpec((q, K_NN * chunks), lambda i: (0, 0)),
        out_shape=jax.ShapeDtypeStruct((q, K_NN * chunks), jnp.int32),
        scratch_shapes=[
            pltpu.VMEM((q, d), jnp.bfloat16),
            pltpu.VMEM((q, K_NN * LANES), jnp.float32),
            pltpu.VMEM((q, K_NN * LANES), jnp.int32),
        ],
        compiler_params=pltpu.CompilerParams(
            dimension_semantics=("arbitrary",),
        ),
        interpret=interpret,
    )(source_feats, target_feats)


def gather_mean(target_feats, idx):
    q = idx.shape[0]
    t, d = target_feats.shape
    chunks = d // 128
    w = 128
    rows_out = w // K_NN
    tgt_view = target_feats.reshape(t * chunks, 128)
    idx_flat = idx.reshape(1, q * K_NN * chunks)
    mesh = plsc.VectorSubcoreMesh(core_axis_name="core", subcore_axis_name="subcore")

    @pl.kernel(
        out_type=jax.ShapeDtypeStruct((q * chunks, 128), jnp.float32),
        mesh=mesh,
        scratch_types=[pltpu.VMEM((w, 128), jnp.float32)],
    )
    def sc_kernel(tgt_hbm, idx_hbm, out_hbm, g_vmem):
        def body(i_vmem, o_vmem):
            pltpu.sync_copy(tgt_hbm.at[i_vmem.at[0]], g_vmem)

            @pl.loop(0, rows_out)
            def _row(r):
                @pl.loop(0, 128, step=16)
                def _col(c):
                    acc = (
                        g_vmem[K_NN * r, pl.ds(c, 16)]
                        + g_vmem[K_NN * r + 1, pl.ds(c, 16)]
                        + g_vmem[K_NN * r + 2, pl.ds(c, 16)]
                        + g_vmem[K_NN * r + 3, pl.ds(c, 16)]
                    )
                    o_vmem[r, pl.ds(c, 16)] = acc * 0.25

        pltpu.emit_pipeline(
            body,
            grid=(q * K_NN * chunks // w,),
            in_specs=[pl.BlockSpec((1, w), lambda i: (0, i))],
            out_specs=[pl.BlockSpec((rows_out, 128), lambda i: (i, 0))],
            core_axis_name=("core", "subcore"),
            dimension_semantics=(pltpu.PARALLEL,),
        )(idx_hbm, out_hbm)

    return sc_kernel(tgt_view, idx_flat).reshape(q, d)


def kernel(source_feats, target_feats):
    idx = topk_indices(source_feats, target_feats)
    return gather_mean(target_feats, idx)
